# Initial kernel scaffold; baseline (speedup 1.0000x reference)
import functools

import jax
import jax.numpy as jnp
from jax import lax
from jax.experimental import pallas as pl
from jax.experimental.pallas import tpu as pltpu
from jax.experimental.pallas import tpu_sc as plsc

N = 10000
D = 256
E = 160000
H = D // 2

NC, NS = 2, 16
CHUNK = 128
CPS = 80
EPAD = NS * CPS * CHUNK
ACC_ROWS = 10240
TRASH = N
ZROWS = 128

RB = 1000
NB = N // RB

_mesh = plsc.VectorSubcoreMesh(core_axis_name="c", subcore_axis_name="s")


def _sc_deg(dst3):
    @functools.partial(
        pl.kernel,
        out_type=jax.ShapeDtypeStruct((N, 16), jnp.float32),
        mesh=_mesh,
        scratch_types=[
            pltpu.VMEM((CPS, CHUNK), jnp.int32),
            pltpu.VMEM((CHUNK, 16), jnp.float32),
            pltpu.VMEM((ZROWS, 16), jnp.float32),
            pltpu.VMEM_SHARED((ACC_ROWS, 16), jnp.float32),
        ],
    )
    def k(dst_hbm, out_hbm, didx, ones, zbuf, acc):
        cid = lax.axis_index("c")
        sid = lax.axis_index("s")

        @pl.loop(0, CHUNK)
        def _(i):
            ones[i, :] = jnp.ones((16,), jnp.float32)

        @pl.loop(0, ZROWS)
        def _(i):
            zbuf[i, :] = jnp.zeros((16,), jnp.float32)

        rows_per = ACC_ROWS // NS
        @pl.loop(0, rows_per, step=ZROWS)
        def _(r):
            pltpu.sync_copy(zbuf, acc.at[pl.ds(sid * rows_per + r, ZROWS)])
        plsc.subcore_barrier()

        pltpu.sync_copy(dst_hbm.at[sid], didx)

        @pl.loop(0, CPS)
        def _(j):
            pltpu.sync_copy(ones, acc.at[didx.at[j]], add=True)
        plsc.subcore_barrier()

        @pl.when(cid == 0)
        def _():
            out_rows = N // NS
            base = sid * out_rows
            pltpu.sync_copy(acc.at[pl.ds(base, out_rows)],
                            out_hbm.at[pl.ds(base, out_rows)])

    return k(dst3)


def _sc_scatter(hp, src3, dst3):
    @functools.partial(
        pl.kernel,
        out_type=jax.ShapeDtypeStruct((NC, N, H), jnp.float32),
        mesh=_mesh,
        scratch_types=[
            pltpu.VMEM((CPS, CHUNK), jnp.int32),
            pltpu.VMEM((CPS, CHUNK), jnp.int32),
            pltpu.VMEM((2, CHUNK, H), jnp.float32),
            pltpu.VMEM((ZROWS, H), jnp.float32),
            pltpu.VMEM_SHARED((ACC_ROWS, H), jnp.float32),
            pltpu.SemaphoreType.DMA,
        ],
    )
    def k(hp_hbm, src_hbm, dst_hbm, out_hbm, sidx, didx, gbuf, zbuf, acc, gsem):
        cid = lax.axis_index("c")
        sid = lax.axis_index("s")

        @pl.loop(0, ZROWS)
        def _(i):
            @pl.loop(0, H, step=16)
            def _(c):
                zbuf[i, pl.ds(c, 16)] = jnp.zeros((16,), jnp.float32)

        rows_per = ACC_ROWS // NS
        @pl.loop(0, rows_per, step=ZROWS)
        def _(r):
            pltpu.sync_copy(zbuf, acc.at[pl.ds(sid * rows_per + r, ZROWS)])
        plsc.subcore_barrier()

        pltpu.sync_copy(src_hbm.at[sid], sidx)
        pltpu.sync_copy(dst_hbm.at[sid], didx)

        tbl = hp_hbm.at[cid]

        def gather(j, b):
            return pltpu.make_async_copy(tbl.at[sidx.at[j]], gbuf.at[b], gsem)

        gather(0, 0).start()
        gather(1, 1).start()

        @pl.loop(0, CPS - 2, step=2)
        def _(j):
            for b in range(2):
                gather(j + b, b).wait()
                pltpu.sync_copy(gbuf.at[b], acc.at[didx.at[j + b]], add=True)
                gather(j + b + 2, b).start()

        for b in range(2):
            gather(CPS - 2 + b, b).wait()
            pltpu.sync_copy(gbuf.at[b], acc.at[didx.at[CPS - 2 + b]], add=True)

        plsc.subcore_barrier()

        out_rows = N // NS
        base = sid * out_rows
        pltpu.sync_copy(acc.at[pl.ds(base, out_rows)],
                        out_hbm.at[cid].at[pl.ds(base, out_rows)])

    return k(hp, src3, dst3)


def _dot(a, b):
    return jnp.dot(a, b, preferred_element_type=jnp.float32,
                   precision=lax.Precision.HIGHEST)


def _tc_first(x, deg, w):
    def body(x_ref, deg_ref, w_ref, out_ref):
        dis = lax.rsqrt(deg_ref[:, :1] + 1.0)
        hp = _dot(x_ref[...], w_ref[...]) * dis
        out_ref[0] = hp[:, :H]
        out_ref[1] = hp[:, H:]

    return pl.pallas_call(
        body,
        grid=(NB,),
        in_specs=[
            pl.BlockSpec((RB, D), lambda i: (i, 0)),
            pl.BlockSpec((RB, 16), lambda i: (i, 0)),
            pl.BlockSpec((D, D), lambda i: (0, 0)),
        ],
        out_specs=pl.BlockSpec((NC, RB, H), lambda i: (0, i, 0)),
        out_shape=jax.ShapeDtypeStruct((NC, N, H), jnp.float32),
    )(x, deg, w)


def _tc_layer(s, hp, deg, b, w):
    def body(s_ref, hp_ref, deg_ref, b_ref, w_ref, out_ref):
        dis = lax.rsqrt(deg_ref[:, :1] + 1.0)
        agg = jnp.concatenate(
            [s_ref[0] + hp_ref[0], s_ref[1] + hp_ref[1]], axis=1)
        z = jnp.maximum(agg * dis + b_ref[0], 0.0)
        hpn = _dot(z, w_ref[...]) * dis
        out_ref[0] = hpn[:, :H]
        out_ref[1] = hpn[:, H:]

    return pl.pallas_call(
        body,
        grid=(NB,),
        in_specs=[
            pl.BlockSpec((NC, RB, H), lambda i: (0, i, 0)),
            pl.BlockSpec((NC, RB, H), lambda i: (0, i, 0)),
            pl.BlockSpec((RB, 16), lambda i: (i, 0)),
            pl.BlockSpec((1, D), lambda i: (0, 0)),
            pl.BlockSpec((D, D), lambda i: (0, 0)),
        ],
        out_specs=pl.BlockSpec((NC, RB, H), lambda i: (0, i, 0)),
        out_shape=jax.ShapeDtypeStruct((NC, N, H), jnp.float32),
    )(s, hp, deg, b, w)


def _tc_final(s, hp, deg, b):
    def body(s_ref, hp_ref, deg_ref, b_ref, out_ref):
        dis = lax.rsqrt(deg_ref[:, :1] + 1.0)
        agg = jnp.concatenate(
            [s_ref[0] + hp_ref[0], s_ref[1] + hp_ref[1]], axis=1)
        out_ref[...] = agg * dis + b_ref[0]

    return pl.pallas_call(
        body,
        grid=(NB,),
        in_specs=[
            pl.BlockSpec((NC, RB, H), lambda i: (0, i, 0)),
            pl.BlockSpec((NC, RB, H), lambda i: (0, i, 0)),
            pl.BlockSpec((RB, 16), lambda i: (i, 0)),
            pl.BlockSpec((1, D), lambda i: (0, 0)),
        ],
        out_specs=pl.BlockSpec((RB, D), lambda i: (i, 0)),
        out_shape=jax.ShapeDtypeStruct((N, D), jnp.float32),
    )(s, hp, deg, b)


def kernel(x, edge_index, W1, b1, W2, b2, W3, b3):
    src = edge_index[0]
    dst = edge_index[1]
    pad = EPAD - E
    srcp = jnp.concatenate([src, jnp.zeros((pad,), jnp.int32)])
    dstp = jnp.concatenate([dst, jnp.full((pad,), TRASH, jnp.int32)])
    src3 = srcp.reshape(NS, CPS, CHUNK)
    dst3 = dstp.reshape(NS, CPS, CHUNK)
    b1r, b2r, b3r = (v.reshape(1, D) for v in (b1, b2, b3))

    deg = _sc_deg(dst3)
    hp1 = _tc_first(x, deg, W1)
    s1 = _sc_scatter(hp1, src3, dst3)
    hp2 = _tc_layer(s1, hp1, deg, b1r, W2)
    s2 = _sc_scatter(hp2, src3, dst3)
    hp3 = _tc_layer(s2, hp2, deg, b2r, W3)
    s3 = _sc_scatter(hp3, src3, dst3)
    return _tc_final(s3, hp3, deg, b3r)


# speedup vs baseline: 7.9543x; 7.9543x over previous
import functools

import jax
import jax.numpy as jnp
from jax import lax
from jax.experimental import pallas as pl
from jax.experimental.pallas import tpu as pltpu
from jax.experimental.pallas import tpu_sc as plsc

N = 10000
D = 256
E = 160000
H = D // 2

NC, NS = 2, 16
CHUNK = 128
CPS = 80
SUP = 4
CPW = CPS // SUP
EPAD = NS * CPS * CHUNK
ACC_ROWS = 10240
TRASH = N
ZROWS = 128

RB = 1000
NB = N // RB

_mesh = plsc.VectorSubcoreMesh(core_axis_name="c", subcore_axis_name="s")


def _sc_deg(dst3):
    @functools.partial(
        pl.kernel,
        out_type=jax.ShapeDtypeStruct((NC, N, 16), jnp.float32),
        mesh=_mesh,
        scratch_types=[
            pltpu.VMEM((CPS, CHUNK), jnp.int32),
            pltpu.VMEM((CHUNK, 16), jnp.float32),
            pltpu.VMEM((ZROWS, 16), jnp.float32),
            pltpu.VMEM_SHARED((ACC_ROWS, 16), jnp.float32),
        ],
    )
    def k(dst_hbm, out_hbm, didx, ones, zbuf, acc):
        cid = lax.axis_index("c")
        sid = lax.axis_index("s")

        @pl.loop(0, CHUNK)
        def _(i):
            ones[i, :] = jnp.ones((16,), jnp.float32)

        @pl.loop(0, ZROWS)
        def _(i):
            zbuf[i, :] = jnp.zeros((16,), jnp.float32)

        rows_per = ACC_ROWS // NS
        @pl.loop(0, rows_per, step=ZROWS)
        def _(r):
            pltpu.sync_copy(zbuf, acc.at[pl.ds(sid * rows_per + r, ZROWS)])
        plsc.subcore_barrier()

        pltpu.sync_copy(dst_hbm.at[sid], didx)

        @pl.loop(0, CPS // NC)
        def _(j):
            pltpu.sync_copy(ones, acc.at[didx.at[cid * (CPS // NC) + j]],
                            add=True)
        plsc.subcore_barrier()

        @pl.when(sid < NS - 1)
        def _():
            base = sid * 624
            pltpu.sync_copy(acc.at[pl.ds(base, 624)],
                            out_hbm.at[cid].at[pl.ds(base, 624)])

        @pl.when(sid == NS - 1)
        def _():
            pltpu.sync_copy(acc.at[pl.ds(9360, 640)],
                            out_hbm.at[cid].at[pl.ds(9360, 640)])

    return k(dst3)


def _sc_scatter(hp, src4, dst4):
    @functools.partial(
        pl.kernel,
        out_type=jax.ShapeDtypeStruct((NC, N, H), jnp.float32),
        mesh=_mesh,
        scratch_types=[
            pltpu.VMEM((CPW, CHUNK), jnp.int32),
            pltpu.VMEM((CPW, CHUNK), jnp.int32),
            pltpu.VMEM((2, CHUNK, H), jnp.float32),
            pltpu.VMEM_SHARED((ACC_ROWS, H), jnp.float32),
            pltpu.SemaphoreType.DMA,
            pltpu.SemaphoreType.DMA,
        ],
    )
    def k(hp_hbm, src_hbm, dst_hbm, out_hbm, sidx, didx, gbuf, acc,
          gsem0, gsem1):
        gsems = (gsem0, gsem1)
        cid = lax.axis_index("c")
        sid = lax.axis_index("s")

        for b in range(2):
            @pl.loop(0, ZROWS)
            def _(i):
                @pl.loop(0, H, step=16)
                def _(c):
                    gbuf[b, i, pl.ds(c, 16)] = jnp.zeros((16,), jnp.float32)

        rows_per = ACC_ROWS // NS
        @pl.loop(0, rows_per, step=ZROWS)
        def _(r):
            pltpu.sync_copy(gbuf.at[0],
                            acc.at[pl.ds(sid * rows_per + r, ZROWS)])
        plsc.subcore_barrier()

        tbl = hp_hbm.at[cid]

        def gather(j, b):
            return pltpu.make_async_copy(tbl.at[sidx.at[j]], gbuf.at[b],
                                         gsems[b])

        @pl.loop(0, SUP)
        def _(w):
            pltpu.sync_copy(src_hbm.at[sid].at[w], sidx)
            pltpu.sync_copy(dst_hbm.at[sid].at[w], didx)

            gather(0, 0).start()
            gather(1, 1).start()

            @pl.loop(0, CPW - 2, step=2)
            def _(j):
                for b in range(2):
                    gather(j + b, b).wait()
                    pltpu.sync_copy(gbuf.at[b], acc.at[didx.at[j + b]],
                                    add=True)
                    gather(j + b + 2, b).start()

            for b in range(2):
                gather(CPW - 2 + b, b).wait()
                pltpu.sync_copy(gbuf.at[b], acc.at[didx.at[CPW - 2 + b]],
                                add=True)

        plsc.subcore_barrier()

        @pl.when(sid < NS - 1)
        def _():
            base = sid * 624
            pltpu.sync_copy(acc.at[pl.ds(base, 624)],
                            out_hbm.at[cid].at[pl.ds(base, 624)])

        @pl.when(sid == NS - 1)
        def _():
            pltpu.sync_copy(acc.at[pl.ds(9360, 640)],
                            out_hbm.at[cid].at[pl.ds(9360, 640)])

    return k(hp, src4, dst4)


def _dot(a, b):
    return jnp.dot(a, b, preferred_element_type=jnp.float32,
                   precision=lax.Precision.HIGHEST)


def _tc_first(x, deg, w):
    def body(x_ref, deg_ref, w_ref, out_ref):
        dis = lax.rsqrt(deg_ref[0, :, :1] + deg_ref[1, :, :1] + 1.0)
        hp = _dot(x_ref[...], w_ref[...]) * dis
        out_ref[0] = hp[:, :H]
        out_ref[1] = hp[:, H:]

    return pl.pallas_call(
        body,
        grid=(NB,),
        in_specs=[
            pl.BlockSpec((RB, D), lambda i: (i, 0)),
            pl.BlockSpec((NC, RB, 16), lambda i: (0, i, 0)),
            pl.BlockSpec((D, D), lambda i: (0, 0)),
        ],
        out_specs=pl.BlockSpec((NC, RB, H), lambda i: (0, i, 0)),
        out_shape=jax.ShapeDtypeStruct((NC, N, H), jnp.float32),
    )(x, deg, w)


def _tc_layer(s, hp, deg, b, w):
    def body(s_ref, hp_ref, deg_ref, b_ref, w_ref, out_ref):
        dis = lax.rsqrt(deg_ref[0, :, :1] + deg_ref[1, :, :1] + 1.0)
        agg = jnp.concatenate(
            [s_ref[0] + hp_ref[0], s_ref[1] + hp_ref[1]], axis=1)
        z = jnp.maximum(agg * dis + b_ref[0], 0.0)
        hpn = _dot(z, w_ref[...]) * dis
        out_ref[0] = hpn[:, :H]
        out_ref[1] = hpn[:, H:]

    return pl.pallas_call(
        body,
        grid=(NB,),
        in_specs=[
            pl.BlockSpec((NC, RB, H), lambda i: (0, i, 0)),
            pl.BlockSpec((NC, RB, H), lambda i: (0, i, 0)),
            pl.BlockSpec((NC, RB, 16), lambda i: (0, i, 0)),
            pl.BlockSpec((1, D), lambda i: (0, 0)),
            pl.BlockSpec((D, D), lambda i: (0, 0)),
        ],
        out_specs=pl.BlockSpec((NC, RB, H), lambda i: (0, i, 0)),
        out_shape=jax.ShapeDtypeStruct((NC, N, H), jnp.float32),
    )(s, hp, deg, b, w)


def _tc_final(s, hp, deg, b):
    def body(s_ref, hp_ref, deg_ref, b_ref, out_ref):
        dis = lax.rsqrt(deg_ref[0, :, :1] + deg_ref[1, :, :1] + 1.0)
        agg = jnp.concatenate(
            [s_ref[0] + hp_ref[0], s_ref[1] + hp_ref[1]], axis=1)
        out_ref[...] = agg * dis + b_ref[0]

    return pl.pallas_call(
        body,
        grid=(NB,),
        in_specs=[
            pl.BlockSpec((NC, RB, H), lambda i: (0, i, 0)),
            pl.BlockSpec((NC, RB, H), lambda i: (0, i, 0)),
            pl.BlockSpec((NC, RB, 16), lambda i: (0, i, 0)),
            pl.BlockSpec((1, D), lambda i: (0, 0)),
        ],
        out_specs=pl.BlockSpec((RB, D), lambda i: (i, 0)),
        out_shape=jax.ShapeDtypeStruct((N, D), jnp.float32),
    )(s, hp, deg, b)


def kernel(x, edge_index, W1, b1, W2, b2, W3, b3):
    src = edge_index[0]
    dst = edge_index[1]
    pad = EPAD - E
    srcp = jnp.concatenate([src, jnp.zeros((pad,), jnp.int32)])
    dstp = jnp.concatenate([dst, jnp.full((pad,), TRASH, jnp.int32)])
    src3 = srcp.reshape(NS, CPS, CHUNK)
    dst3 = dstp.reshape(NS, CPS, CHUNK)
    src4 = srcp.reshape(NS, SUP, CPW, CHUNK)
    dst4 = dstp.reshape(NS, SUP, CPW, CHUNK)
    b1r, b2r, b3r = (v.reshape(1, D) for v in (b1, b2, b3))

    deg = _sc_deg(dst3)
    hp1 = _tc_first(x, deg, W1)
    s1 = _sc_scatter(hp1, src4, dst4)
    hp2 = _tc_layer(s1, hp1, deg, b1r, W2)
    s2 = _sc_scatter(hp2, src4, dst4)
    hp3 = _tc_layer(s2, hp2, deg, b2r, W3)
    s3 = _sc_scatter(hp3, src4, dst4)
    return _tc_final(s3, hp3, deg, b3r)
